# Initial kernel scaffold; baseline (speedup 1.0000x reference)
import jax
import jax.numpy as jnp
from jax.experimental import pallas as pl


def kernel(x, batch, W1, b1, gamma, beta, W2, b2):
    raise NotImplementedError("write your pallas kernel here")


# speedup vs baseline: 1.2072x; 1.2072x over previous
import functools

import jax
import jax.numpy as jnp
from jax import lax
from jax.experimental import pallas as pl
from jax.experimental.pallas import tpu as pltpu
from jax.experimental.pallas import tpu_sc as plsc

N = 100000
D = 128
S = 1024
EPS = 1e-5

NC = 2
NS = 16
NW = NC * NS
CHUNK = 80
NCHUNK = N // CHUNK
ITERS = (NCHUNK + NW - 1) // NW
LPR = D // 16

_mesh = plsc.VectorSubcoreMesh(core_axis_name="c", subcore_axis_name="s")


def _zero_rows(ref, nrows):
    z = jnp.zeros((16,), jnp.float32)

    def body(r, _):
        for k in range(LPR):
            ref[r, pl.ds(k * 16, 16)] = z
        return 0

    lax.fori_loop(0, nrows, body, 0)


@functools.partial(
    pl.kernel,
    out_type=jax.ShapeDtypeStruct((NC * S, D), jnp.float32),
    mesh=_mesh,
    scratch_types=[
        pltpu.VMEM((CHUNK, D), jnp.float32),
        pltpu.VMEM((CHUNK,), jnp.int32),
        pltpu.VMEM_SHARED((S, D), jnp.float32),
    ],
)
def _segsum_k(x_hbm, b_hbm, out_hbm, bufx, idxv, acc):
    cid = lax.axis_index("c")
    sid = lax.axis_index("s")
    wid = sid * NC + cid
    rows_per_tile = S // NS

    _zero_rows(bufx, rows_per_tile)
    pltpu.sync_copy(bufx.at[pl.ds(0, rows_per_tile)],
                    acc.at[pl.ds(sid * rows_per_tile, rows_per_tile)])
    plsc.subcore_barrier()

    def chunk_body(i, _):
        c = wid + i * NW

        @pl.when(c < NCHUNK)
        def _():
            row0 = pl.multiple_of(c * CHUNK, 8)
            pltpu.sync_copy(x_hbm.at[pl.ds(row0, CHUNK)], bufx)
            pltpu.sync_copy(b_hbm.at[pl.ds(row0, CHUNK)], idxv)
            pltpu.sync_copy(bufx, acc.at[idxv], add=True)

        return 0

    lax.fori_loop(0, ITERS, chunk_body, 0)
    plsc.subcore_barrier()

    pltpu.sync_copy(
        acc.at[pl.ds(sid * rows_per_tile, rows_per_tile)],
        out_hbm.at[pl.ds(cid * S + sid * rows_per_tile, rows_per_tile)])


def _mlp_body(hp_ref, w1_ref, b1_ref, g_ref, be_ref, w2_ref, b2_ref, vn_ref):
    h = hp_ref[0] + hp_ref[1]
    z = lax.dot_general(h, w1_ref[...], (((1,), (1,)), ((), ())),
                        preferred_element_type=jnp.float32) + b1_ref[...]
    mu = jnp.mean(z, axis=0, keepdims=True)
    var = jnp.mean(jnp.square(z - mu), axis=0, keepdims=True)
    zn = (z - mu) * lax.rsqrt(var + EPS) * g_ref[...] + be_ref[...]
    a = jnp.maximum(zn, 0.0)
    vn = lax.dot_general(a, w2_ref[...], (((1,), (1,)), ((), ())),
                         preferred_element_type=jnp.float32) + b2_ref[...]
    vn_ref[...] = vn


_mlp = pl.pallas_call(
    _mlp_body,
    out_shape=jax.ShapeDtypeStruct((S, D), jnp.float32),
)


@functools.partial(
    pl.kernel,
    out_type=jax.ShapeDtypeStruct((N, D), jnp.float32),
    mesh=_mesh,
    scratch_types=[
        pltpu.VMEM((CHUNK, D), jnp.float32),
        pltpu.VMEM((CHUNK, D), jnp.float32),
        pltpu.VMEM((CHUNK,), jnp.int32),
    ],
)
def _bcast_k(x_hbm, b_hbm, vn_hbm, out_hbm, bufx, bufv, idxv):
    cid = lax.axis_index("c")
    sid = lax.axis_index("s")
    wid = sid * NC + cid

    def chunk_body(i, _):
        c = wid + i * NW

        @pl.when(c < NCHUNK)
        def _():
            row0 = pl.multiple_of(c * CHUNK, 8)
            pltpu.sync_copy(x_hbm.at[pl.ds(row0, CHUNK)], bufx)
            pltpu.sync_copy(b_hbm.at[pl.ds(row0, CHUNK)], idxv)
            pltpu.sync_copy(vn_hbm.at[idxv], bufv)

            def add_body(r, _):
                for k in range(LPR):
                    bufx[r, pl.ds(k * 16, 16)] = (
                        bufx[r, pl.ds(k * 16, 16)] + bufv[r, pl.ds(k * 16, 16)])
                return 0

            lax.fori_loop(0, CHUNK, add_body, 0)
            pltpu.sync_copy(bufx, out_hbm.at[pl.ds(row0, CHUNK)])

        return 0

    lax.fori_loop(0, ITERS, chunk_body, 0)


def kernel(x, batch, W1, b1, gamma, beta, W2, b2):
    batch32 = batch.astype(jnp.int32)
    hp = _segsum_k(x, batch32).reshape(NC, S, D)
    vn = _mlp(hp, W1, b1.reshape(1, D), gamma.reshape(1, D),
              beta.reshape(1, D), W2, b2.reshape(1, D))
    return _bcast_k(x, batch32, vn)


# speedup vs baseline: 1.8213x; 1.5087x over previous
import functools

import jax
import jax.numpy as jnp
from jax import lax
from jax.experimental import pallas as pl
from jax.experimental.pallas import tpu as pltpu
from jax.experimental.pallas import tpu_sc as plsc

N = 100000
D = 128
S = 1024
EPS = 1e-5

NC = 2
NS = 16
NW = NC * NS
CHUNK = 80
CPW = 39
RPW = CPW * CHUNK
TAIL = NW * RPW
LPR = D // 16

_mesh = plsc.VectorSubcoreMesh(core_axis_name="c", subcore_axis_name="s")


def _restage_idx(idxs, idx80, i):
    for k in range(CHUNK // 16):
        idx80[pl.ds(k * 16, 16)] = idxs[pl.ds(i * CHUNK + k * 16, 16)]


@functools.partial(
    pl.kernel,
    out_type=jax.ShapeDtypeStruct((NC * S, D), jnp.float32),
    mesh=_mesh,
    scratch_types=[
        pltpu.VMEM((CHUNK, D), jnp.float32),
        pltpu.VMEM((CHUNK, D), jnp.float32),
        pltpu.VMEM((RPW,), jnp.int32),
        pltpu.VMEM((CHUNK,), jnp.int32),
        pltpu.VMEM_SHARED((S, D), jnp.float32),
        pltpu.SemaphoreType.DMA,
        pltpu.SemaphoreType.DMA,
    ],
)
def _segsum_k(x_hbm, b_hbm, out_hbm, bufx0, bufx1, idxs, idx80, acc,
              semx0, semx1):
    cid = lax.axis_index("c")
    sid = lax.axis_index("s")
    wid = sid * NC + cid
    r0 = pl.multiple_of(wid * RPW, 8)
    rows_per_tile = S // NS
    bufx = (bufx0, bufx1)
    semx = (semx0, semx1)

    z = jnp.zeros((16,), jnp.float32)

    def zbody(r, _):
        for k in range(LPR):
            bufx0[r, pl.ds(k * 16, 16)] = z
        return 0

    lax.fori_loop(0, rows_per_tile, zbody, 0)
    pltpu.sync_copy(bufx0.at[pl.ds(0, rows_per_tile)],
                    acc.at[pl.ds(sid * rows_per_tile, rows_per_tile)])

    pltpu.sync_copy(b_hbm.at[pl.ds(r0, RPW)], idxs)
    plsc.subcore_barrier()

    def x_load(i, b):
        return pltpu.async_copy(
            x_hbm.at[pl.ds(r0 + i * CHUNK, CHUNK)], bufx[b], semx[b])

    d = x_load(0, 0)
    for i in range(CPW):
        b = i % 2
        d.wait()
        if i + 1 < CPW:
            d = x_load(i + 1, 1 - b)
        _restage_idx(idxs, idx80, i)
        pltpu.sync_copy(bufx[b], acc.at[idx80], add=True)

    @pl.when(wid < 2)
    def _():
        t0 = pl.multiple_of(TAIL + wid * CHUNK, 8)
        pltpu.sync_copy(x_hbm.at[pl.ds(t0, CHUNK)], bufx0)
        pltpu.sync_copy(b_hbm.at[pl.ds(t0, CHUNK)], idx80)
        pltpu.sync_copy(bufx0, acc.at[idx80], add=True)

    plsc.subcore_barrier()
    pltpu.sync_copy(
        acc.at[pl.ds(sid * rows_per_tile, rows_per_tile)],
        out_hbm.at[pl.ds(cid * S + sid * rows_per_tile, rows_per_tile)])


def _mlp_body(hp_ref, w1_ref, b1_ref, g_ref, be_ref, w2_ref, b2_ref, vn_ref):
    h = hp_ref[0] + hp_ref[1]
    z = lax.dot_general(h, w1_ref[...], (((1,), (1,)), ((), ())),
                        preferred_element_type=jnp.float32) + b1_ref[...]
    mu = jnp.mean(z, axis=0, keepdims=True)
    var = jnp.mean(jnp.square(z - mu), axis=0, keepdims=True)
    zn = (z - mu) * lax.rsqrt(var + EPS) * g_ref[...] + be_ref[...]
    a = jnp.maximum(zn, 0.0)
    vn = lax.dot_general(a, w2_ref[...], (((1,), (1,)), ((), ())),
                         preferred_element_type=jnp.float32) + b2_ref[...]
    vn_ref[...] = vn


_mlp = pl.pallas_call(
    _mlp_body,
    out_shape=jax.ShapeDtypeStruct((S, D), jnp.float32),
)


@functools.partial(
    pl.kernel,
    out_type=jax.ShapeDtypeStruct((N, D), jnp.float32),
    mesh=_mesh,
    scratch_types=[
        pltpu.VMEM((CHUNK, D), jnp.float32),
        pltpu.VMEM((CHUNK, D), jnp.float32),
        pltpu.VMEM((RPW,), jnp.int32),
        pltpu.VMEM((CHUNK,), jnp.int32),
        pltpu.SemaphoreType.DMA,
        pltpu.SemaphoreType.DMA,
        pltpu.SemaphoreType.DMA,
        pltpu.SemaphoreType.DMA,
        pltpu.SemaphoreType.DMA,
        pltpu.SemaphoreType.DMA,
    ],
)
def _bcast_k(x_hbm, b_hbm, vn_hbm, out_hbm, bufx0, bufx1, idxs, idx80,
             semx0, semx1, semv0, semv1, semo0, semo1):
    cid = lax.axis_index("c")
    sid = lax.axis_index("s")
    wid = sid * NC + cid
    r0 = pl.multiple_of(wid * RPW, 8)
    bufx = (bufx0, bufx1)
    semx = (semx0, semx1)
    semv = (semv0, semv1)
    semo = (semo0, semo1)

    pltpu.sync_copy(b_hbm.at[pl.ds(r0, RPW)], idxs)

    def rows(i):
        return pl.ds(r0 + i * CHUNK, CHUNK)

    def x_load(i, b):
        return pltpu.async_copy(x_hbm.at[rows(i)], bufx[b], semx[b])

    def v_gather_add(i, b):
        return pltpu.async_copy(
            vn_hbm.at[idxs.at[pl.ds(i * CHUNK, CHUNK)]], bufx[b], semv[b],
            add=True)

    dx = {0: x_load(0, 0)}
    dv = {}
    do = {}
    for i in range(CPW):
        b = i % 2
        if i + 1 < CPW:
            if i >= 1:
                do[1 - b].wait()
            dx[1 - b] = x_load(i + 1, 1 - b)
        dx[b].wait()
        dv[b] = v_gather_add(i, b)
        dv[b].wait()
        do[b] = pltpu.async_copy(bufx[b], out_hbm.at[rows(i)], semo[b])
    do[(CPW - 1) % 2].wait()
    do[CPW % 2].wait()

    @pl.when(wid < 2)
    def _():
        t0 = pl.multiple_of(TAIL + wid * CHUNK, 8)
        pltpu.sync_copy(x_hbm.at[pl.ds(t0, CHUNK)], bufx0)
        pltpu.sync_copy(b_hbm.at[pl.ds(t0, CHUNK)], idx80)
        pltpu.sync_copy(vn_hbm.at[idx80], bufx0, add=True)
        pltpu.sync_copy(bufx0, out_hbm.at[pl.ds(t0, CHUNK)])


def kernel(x, batch, W1, b1, gamma, beta, W2, b2):
    batch32 = batch.astype(jnp.int32)
    hp = _segsum_k(x, batch32).reshape(NC, S, D)
    vn = _mlp(hp, W1, b1.reshape(1, D), gamma.reshape(1, D),
              beta.reshape(1, D), W2, b2.reshape(1, D))
    return _bcast_k(x, batch32, vn)


# speedup vs baseline: 2.6126x; 1.4344x over previous
import functools

import jax
import jax.numpy as jnp
from jax import lax
from jax.experimental import pallas as pl
from jax.experimental.pallas import tpu as pltpu
from jax.experimental.pallas import tpu_sc as plsc

N = 100000
D = 128
S = 1024
EPS = 1e-5

NC = 2
NS = 16
NW = NC * NS
RPW = 3120
TAIL = NW * RPW
TCH = 80
LPR = D // 16

ACH = 80
ACPW = RPW // ACH

UNIT = 104
UPW = RPW // UNIT
SLOTS = 7
XLEAD = 3
GLAG = 3

_mesh = plsc.VectorSubcoreMesh(core_axis_name="c", subcore_axis_name="s")


@functools.partial(
    pl.kernel,
    out_type=jax.ShapeDtypeStruct((NC * S, D), jnp.float32),
    mesh=_mesh,
    scratch_types=[
        pltpu.VMEM((ACH, D), jnp.float32),
        pltpu.VMEM((ACH, D), jnp.float32),
        pltpu.VMEM((RPW,), jnp.int32),
        pltpu.VMEM((ACH,), jnp.int32),
        pltpu.VMEM_SHARED((S, D), jnp.float32),
        pltpu.SemaphoreType.DMA,
        pltpu.SemaphoreType.DMA,
    ],
)
def _segsum_k(x_hbm, b_hbm, out_hbm, bufx0, bufx1, idxs, idx80, acc,
              semx0, semx1):
    cid = lax.axis_index("c")
    sid = lax.axis_index("s")
    wid = sid * NC + cid
    r0 = pl.multiple_of(wid * RPW, 8)
    rows_per_tile = S // NS
    bufx = (bufx0, bufx1)
    semx = (semx0, semx1)

    z = jnp.zeros((16,), jnp.float32)

    def zbody(r, _):
        for k in range(LPR):
            bufx0[r, pl.ds(k * 16, 16)] = z
        return 0

    lax.fori_loop(0, rows_per_tile, zbody, 0)
    pltpu.sync_copy(bufx0.at[pl.ds(0, rows_per_tile)],
                    acc.at[pl.ds(sid * rows_per_tile, rows_per_tile)])

    pltpu.sync_copy(b_hbm.at[pl.ds(r0, RPW)], idxs)
    plsc.subcore_barrier()

    def x_load(i, b):
        return pltpu.async_copy(
            x_hbm.at[pl.ds(r0 + i * ACH, ACH)], bufx[b], semx[b])

    d = x_load(0, 0)
    for i in range(ACPW):
        b = i % 2
        d.wait()
        if i + 1 < ACPW:
            d = x_load(i + 1, 1 - b)
        for k in range(ACH // 16):
            idx80[pl.ds(k * 16, 16)] = idxs[pl.ds(i * ACH + k * 16, 16)]
        pltpu.sync_copy(bufx[b], acc.at[idx80], add=True)

    @pl.when(wid < 2)
    def _():
        t0 = pl.multiple_of(TAIL + wid * TCH, 8)
        pltpu.sync_copy(x_hbm.at[pl.ds(t0, TCH)], bufx0)
        pltpu.sync_copy(b_hbm.at[pl.ds(t0, TCH)], idx80)
        pltpu.sync_copy(bufx0, acc.at[idx80], add=True)

    plsc.subcore_barrier()
    pltpu.sync_copy(
        acc.at[pl.ds(sid * rows_per_tile, rows_per_tile)],
        out_hbm.at[pl.ds(cid * S + sid * rows_per_tile, rows_per_tile)])


def _mlp_body(hp_ref, w1_ref, b1_ref, g_ref, be_ref, w2_ref, b2_ref, vn_ref):
    h = hp_ref[0] + hp_ref[1]
    z = lax.dot_general(h, w1_ref[...], (((1,), (1,)), ((), ())),
                        preferred_element_type=jnp.float32) + b1_ref[...]
    mu = jnp.mean(z, axis=0, keepdims=True)
    var = jnp.mean(jnp.square(z - mu), axis=0, keepdims=True)
    zn = (z - mu) * lax.rsqrt(var + EPS) * g_ref[...] + be_ref[...]
    a = jnp.maximum(zn, 0.0)
    vn = lax.dot_general(a, w2_ref[...], (((1,), (1,)), ((), ())),
                         preferred_element_type=jnp.float32) + b2_ref[...]
    vn_ref[...] = vn


_mlp = pl.pallas_call(
    _mlp_body,
    out_shape=jax.ShapeDtypeStruct((S, D), jnp.float32),
)


@functools.partial(
    pl.kernel,
    out_type=jax.ShapeDtypeStruct((N, D), jnp.float32),
    mesh=_mesh,
    scratch_types=(
        [pltpu.VMEM((UNIT, D), jnp.float32) for _ in range(SLOTS)]
        + [pltpu.VMEM((RPW,), jnp.int32),
           pltpu.VMEM((TCH,), jnp.int32)]
        + [pltpu.SemaphoreType.DMA for _ in range(3 * SLOTS)]
    ),
)
def _bcast_k(x_hbm, b_hbm, vn_hbm, out_hbm, *refs):
    bufs = refs[:SLOTS]
    idxs = refs[SLOTS]
    ixt = refs[SLOTS + 1]
    semx = refs[SLOTS + 2:2 * SLOTS + 2]
    semv = refs[2 * SLOTS + 2:3 * SLOTS + 2]
    semo = refs[3 * SLOTS + 2:4 * SLOTS + 2]
    cid = lax.axis_index("c")
    sid = lax.axis_index("s")
    wid = sid * NC + cid
    r0 = pl.multiple_of(wid * RPW, 8)

    pltpu.sync_copy(b_hbm.at[pl.ds(r0, RPW)], idxs)

    def rows(u):
        return pl.ds(r0 + u * UNIT, UNIT)

    def x_load(u):
        k = u % SLOTS
        return pltpu.async_copy(x_hbm.at[rows(u)], bufs[k], semx[k])

    def v_gather_add(u):
        k = u % SLOTS
        return pltpu.async_copy(
            vn_hbm.at[idxs.at[pl.ds(u * UNIT, UNIT)]], bufs[k], semv[k],
            add=True)

    def store(u):
        k = u % SLOTS
        return pltpu.async_copy(bufs[k], out_hbm.at[rows(u)], semo[k])

    dx, dv, do = {}, {}, {}
    for u in range(min(XLEAD, UPW)):
        dx[u % SLOTS] = x_load(u)
    for u in range(UPW):
        k = u % SLOTS
        if u + XLEAD < UPW:
            kn = (u + XLEAD) % SLOTS
            if u >= XLEAD + 1:
                do.pop(kn).wait()
            dx[kn] = x_load(u + XLEAD)
        dx.pop(k).wait()
        dv[k] = v_gather_add(u)
        if u >= GLAG:
            kr = (u - GLAG) % SLOTS
            dv.pop(kr).wait()
            do[kr] = store(u - GLAG)
    for u in range(max(0, UPW - GLAG), UPW):
        kr = u % SLOTS
        dv.pop(kr).wait()
        do[kr] = store(u)
    for dd in do.values():
        dd.wait()

    @pl.when(wid < 2)
    def _():
        t0 = pl.multiple_of(TAIL + wid * TCH, 8)
        pltpu.sync_copy(x_hbm.at[pl.ds(t0, TCH)], bufs[0].at[pl.ds(0, TCH)])
        pltpu.sync_copy(b_hbm.at[pl.ds(t0, TCH)], ixt)
        pltpu.sync_copy(vn_hbm.at[ixt], bufs[0].at[pl.ds(0, TCH)], add=True)
        pltpu.sync_copy(bufs[0].at[pl.ds(0, TCH)], out_hbm.at[pl.ds(t0, TCH)])


def kernel(x, batch, W1, b1, gamma, beta, W2, b2):
    batch32 = batch.astype(jnp.int32)
    hp = _segsum_k(x, batch32).reshape(NC, S, D)
    vn = _mlp(hp, W1, b1.reshape(1, D), gamma.reshape(1, D),
              beta.reshape(1, D), W2, b2.reshape(1, D))
    return _bcast_k(x, batch32, vn)


# speedup vs baseline: 2.6231x; 1.0040x over previous
import functools

import jax
import jax.numpy as jnp
from jax import lax
from jax.experimental import pallas as pl
from jax.experimental.pallas import tpu as pltpu
from jax.experimental.pallas import tpu_sc as plsc

N = 100000
D = 128
S = 1024
EPS = 1e-5

NC = 2
NS = 16
NW = NC * NS
RPW = 3120
TAIL = NW * RPW
TCH = 80
LPR = D // 16

ACH = 80
ACPW = RPW // ACH

UNIT = 104
UPW = RPW // UNIT
SLOTS = 9
XLEAD = 3
GLAG = 5

_mesh = plsc.VectorSubcoreMesh(core_axis_name="c", subcore_axis_name="s")


@functools.partial(
    pl.kernel,
    out_type=jax.ShapeDtypeStruct((NC * S, D), jnp.float32),
    mesh=_mesh,
    scratch_types=[
        pltpu.VMEM((ACH, D), jnp.float32),
        pltpu.VMEM((ACH, D), jnp.float32),
        pltpu.VMEM((RPW,), jnp.int32),
        pltpu.VMEM((ACH,), jnp.int32),
        pltpu.VMEM_SHARED((S, D), jnp.float32),
        pltpu.SemaphoreType.DMA,
        pltpu.SemaphoreType.DMA,
    ],
)
def _segsum_k(x_hbm, b_hbm, out_hbm, bufx0, bufx1, idxs, idx80, acc,
              semx0, semx1):
    cid = lax.axis_index("c")
    sid = lax.axis_index("s")
    wid = sid * NC + cid
    r0 = pl.multiple_of(wid * RPW, 8)
    rows_per_tile = S // NS
    bufx = (bufx0, bufx1)
    semx = (semx0, semx1)

    z = jnp.zeros((16,), jnp.float32)

    def zbody(r, _):
        for k in range(LPR):
            bufx0[r, pl.ds(k * 16, 16)] = z
        return 0

    lax.fori_loop(0, rows_per_tile, zbody, 0)
    pltpu.sync_copy(bufx0.at[pl.ds(0, rows_per_tile)],
                    acc.at[pl.ds(sid * rows_per_tile, rows_per_tile)])

    pltpu.sync_copy(b_hbm.at[pl.ds(r0, RPW)], idxs)
    plsc.subcore_barrier()

    def x_load(i, b):
        return pltpu.async_copy(
            x_hbm.at[pl.ds(r0 + i * ACH, ACH)], bufx[b], semx[b])

    d = x_load(0, 0)
    for i in range(ACPW):
        b = i % 2
        d.wait()
        if i + 1 < ACPW:
            d = x_load(i + 1, 1 - b)
        for k in range(ACH // 16):
            idx80[pl.ds(k * 16, 16)] = idxs[pl.ds(i * ACH + k * 16, 16)]
        pltpu.sync_copy(bufx[b], acc.at[idx80], add=True)

    @pl.when(wid < 2)
    def _():
        t0 = pl.multiple_of(TAIL + wid * TCH, 8)
        pltpu.sync_copy(x_hbm.at[pl.ds(t0, TCH)], bufx0)
        pltpu.sync_copy(b_hbm.at[pl.ds(t0, TCH)], idx80)
        pltpu.sync_copy(bufx0, acc.at[idx80], add=True)

    plsc.subcore_barrier()
    pltpu.sync_copy(
        acc.at[pl.ds(sid * rows_per_tile, rows_per_tile)],
        out_hbm.at[pl.ds(cid * S + sid * rows_per_tile, rows_per_tile)])


def _mlp_body(hp_ref, w1_ref, b1_ref, g_ref, be_ref, w2_ref, b2_ref, vn_ref):
    h = hp_ref[0] + hp_ref[1]
    z = lax.dot_general(h, w1_ref[...], (((1,), (1,)), ((), ())),
                        preferred_element_type=jnp.float32) + b1_ref[...]
    mu = jnp.mean(z, axis=0, keepdims=True)
    var = jnp.mean(jnp.square(z - mu), axis=0, keepdims=True)
    zn = (z - mu) * lax.rsqrt(var + EPS) * g_ref[...] + be_ref[...]
    a = jnp.maximum(zn, 0.0)
    vn = lax.dot_general(a, w2_ref[...], (((1,), (1,)), ((), ())),
                         preferred_element_type=jnp.float32) + b2_ref[...]
    vn_ref[...] = vn


_mlp = pl.pallas_call(
    _mlp_body,
    out_shape=jax.ShapeDtypeStruct((S, D), jnp.float32),
)


@functools.partial(
    pl.kernel,
    out_type=jax.ShapeDtypeStruct((N, D), jnp.float32),
    mesh=_mesh,
    scratch_types=(
        [pltpu.VMEM((UNIT, D), jnp.float32) for _ in range(SLOTS)]
        + [pltpu.VMEM((RPW,), jnp.int32),
           pltpu.VMEM((TCH,), jnp.int32)]
        + [pltpu.SemaphoreType.DMA for _ in range(3 * SLOTS)]
    ),
)
def _bcast_k(x_hbm, b_hbm, vn_hbm, out_hbm, *refs):
    bufs = refs[:SLOTS]
    idxs = refs[SLOTS]
    ixt = refs[SLOTS + 1]
    semx = refs[SLOTS + 2:2 * SLOTS + 2]
    semv = refs[2 * SLOTS + 2:3 * SLOTS + 2]
    semo = refs[3 * SLOTS + 2:4 * SLOTS + 2]
    cid = lax.axis_index("c")
    sid = lax.axis_index("s")
    wid = sid * NC + cid
    r0 = pl.multiple_of(wid * RPW, 8)

    pltpu.sync_copy(b_hbm.at[pl.ds(r0, RPW)], idxs)

    def rows(u):
        return pl.ds(r0 + u * UNIT, UNIT)

    def x_load(u):
        k = u % SLOTS
        return pltpu.async_copy(x_hbm.at[rows(u)], bufs[k], semx[k])

    def v_gather_add(u):
        k = u % SLOTS
        return pltpu.async_copy(
            vn_hbm.at[idxs.at[pl.ds(u * UNIT, UNIT)]], bufs[k], semv[k],
            add=True)

    def store(u):
        k = u % SLOTS
        return pltpu.async_copy(bufs[k], out_hbm.at[rows(u)], semo[k])

    dx, dv, do = {}, {}, {}
    for u in range(min(XLEAD, UPW)):
        dx[u % SLOTS] = x_load(u)
    for u in range(UPW):
        k = u % SLOTS
        if u + XLEAD < UPW:
            kn = (u + XLEAD) % SLOTS
            if u >= SLOTS - XLEAD:
                do.pop(kn).wait()
            dx[kn] = x_load(u + XLEAD)
        dx.pop(k).wait()
        dv[k] = v_gather_add(u)
        if u >= GLAG:
            kr = (u - GLAG) % SLOTS
            dv.pop(kr).wait()
            do[kr] = store(u - GLAG)
    for u in range(max(0, UPW - GLAG), UPW):
        kr = u % SLOTS
        dv.pop(kr).wait()
        do[kr] = store(u)
    for dd in do.values():
        dd.wait()

    @pl.when(wid < 2)
    def _():
        t0 = pl.multiple_of(TAIL + wid * TCH, 8)
        pltpu.sync_copy(x_hbm.at[pl.ds(t0, TCH)], bufs[0].at[pl.ds(0, TCH)])
        pltpu.sync_copy(b_hbm.at[pl.ds(t0, TCH)], ixt)
        pltpu.sync_copy(vn_hbm.at[ixt], bufs[0].at[pl.ds(0, TCH)], add=True)
        pltpu.sync_copy(bufs[0].at[pl.ds(0, TCH)], out_hbm.at[pl.ds(t0, TCH)])


def kernel(x, batch, W1, b1, gamma, beta, W2, b2):
    batch32 = batch.astype(jnp.int32)
    hp = _segsum_k(x, batch32).reshape(NC, S, D)
    vn = _mlp(hp, W1, b1.reshape(1, D), gamma.reshape(1, D),
              beta.reshape(1, D), W2, b2.reshape(1, D))
    return _bcast_k(x, batch32, vn)
